# baseline (speedup 1.0000x reference)
import jax
import jax.numpy as jnp
from jax import lax
from jax.experimental import pallas as pl
from jax.experimental.pallas import tpu as pltpu
from jax.experimental.pallas import tpu_sc as plsc

VOCAB = 50257
D = 768
BATCH = 4
SEQ = 2048
NC = 2
NS = 16
NW = NC * NS
PPW = SEQ // NW
C = 16
QUARTERS = PPW // C
NCHUNK = BATCH * QUARTERS
LANES = 16
VECS_PER_ROW = D // LANES


def _emb_body(ids_hbm, wte_hbm, wpe_hbm, out_hbm,
              idx_v, wpe_v, r0, r1, r2, r3,
              g0, g1, g2, g3, s0, s1, s2, s3, wsem):
    rows = [r0, r1, r2, r3]
    gsems = [g0, g1, g2, g3]
    ssems = [s0, s1, s2, s3]

    wid = lax.axis_index("s") * NC + lax.axis_index("c")
    pos_base = wid * PPW

    wdesc = pltpu.async_copy(wpe_hbm.at[pl.ds(pos_base, PPW)], wpe_v, wsem)
    for b in range(BATCH):
        pltpu.sync_copy(ids_hbm.at[b, pl.ds(pos_base, PPW)],
                        idx_v.at[pl.ds(b * PPW, PPW)])

    def issue_gather(ci, q):
        pltpu.async_copy(
            wte_hbm.at[idx_v.at[pl.ds(ci * C, C)]], rows[q], gsems[q])

    issue_gather(0, 0)
    issue_gather(1, 1)
    wdesc.wait()

    def chunk_body(ci, carry):
        nxt = ci + 2
        for q in range(QUARTERS):
            @pl.when(jnp.logical_and(nxt < NCHUNK, (nxt % QUARTERS) == q))
            def _():
                @pl.when(ci >= 2)
                def _():
                    pltpu.make_async_copy(
                        rows[q],
                        out_hbm.at[0, pl.ds(pos_base + q * C, C)],
                        ssems[q]).wait()
                issue_gather(nxt, q)

        for q in range(QUARTERS):
            @pl.when((ci % QUARTERS) == q)
            def _():
                b = ci // QUARTERS
                dst = out_hbm.at[b, pl.ds(pos_base + q * C, C)]
                pltpu.make_async_copy(wte_hbm.at[idx_v.at[pl.ds(0, C)]],
                                      rows[q], gsems[q]).wait()
                rbuf = rows[q]

                @plsc.parallel_loop(0, C, unroll=2)
                def add_row(r):
                    for j in range(VECS_PER_ROW):
                        off = j * LANES
                        v = wpe_v[q * C + r, pl.ds(off, LANES)]
                        plsc.addupdate(rbuf.at[r, pl.ds(off, LANES)], v)

                pltpu.async_copy(rbuf, dst, ssems[q])
        return carry

    lax.fori_loop(0, NCHUNK, chunk_body, 0)

    for q in range(QUARTERS):
        pltpu.make_async_copy(
            rows[q], out_hbm.at[0, pl.ds(pos_base + q * C, C)],
            ssems[q]).wait()


@jax.jit
def _emb(ids, wte, wpe):
    mesh = plsc.VectorSubcoreMesh(
        core_axis_name="c", subcore_axis_name="s", num_cores=NC, num_subcores=NS
    )
    return pl.kernel(
        _emb_body,
        out_type=jax.ShapeDtypeStruct((BATCH, SEQ, D), jnp.float32),
        mesh=mesh,
        scratch_types=[
            pltpu.VMEM((BATCH * PPW,), jnp.int32),
            pltpu.VMEM((PPW, D), jnp.float32),
            pltpu.VMEM((C, D), jnp.float32),
            pltpu.VMEM((C, D), jnp.float32),
            pltpu.VMEM((C, D), jnp.float32),
            pltpu.VMEM((C, D), jnp.float32),
            pltpu.SemaphoreType.DMA,
            pltpu.SemaphoreType.DMA,
            pltpu.SemaphoreType.DMA,
            pltpu.SemaphoreType.DMA,
            pltpu.SemaphoreType.DMA,
            pltpu.SemaphoreType.DMA,
            pltpu.SemaphoreType.DMA,
            pltpu.SemaphoreType.DMA,
            pltpu.SemaphoreType.DMA,
        ],
    )(ids, wte, wpe)


def kernel(input_ids, wte, wpe):
    return _emb(input_ids, wte, wpe)
